# Initial kernel scaffold; baseline (speedup 1.0000x reference)
import jax
import jax.numpy as jnp
from jax.experimental import pallas as pl


def kernel(features, conv_w, conv_b, cls_w, cls_b, bbox_w, bbox_b):
    raise NotImplementedError("write your pallas kernel here")


# speedup vs baseline: 10.1066x; 10.1066x over previous
import math

import jax
import jax.numpy as jnp
import numpy as np
from jax.experimental import pallas as pl

IN_CH = 256
STRIDE = 16
SIZES = (32, 64, 128, 256, 512)
RATIOS = (0.5, 1.0, 2.0)
A = len(SIZES) * len(RATIOS)
IMG_H = 800
IMG_W = 800
FH = 50
FW = 50
PRE_NMS = 2000
POST_NMS = 1000
NMS_TH = 0.7
BBOX_XFORM_CLIP = math.log(1000.0 / 16)

NPAD = 2048
NR = 16
NC = 128


def _base_anchors_np():
    w = float(STRIDE)
    h = float(STRIDE)
    x_ctr = 0.5 * (w - 1)
    y_ctr = 0.5 * (h - 1)
    size = w * h
    anchors = []
    for r in RATIOS:
        ws = round(math.sqrt(size / r))
        hs = round(ws * r)
        for s in SIZES:
            sc = s / STRIDE
            W_ = ws * sc
            H_ = hs * sc
            anchors.append([x_ctr - 0.5 * (W_ - 1), y_ctr - 0.5 * (H_ - 1),
                            x_ctr + 0.5 * (W_ - 1), y_ctr + 0.5 * (H_ - 1)])
    return np.array(anchors, dtype=np.float32)


def _grid_anchors_np():
    base = _base_anchors_np()
    sx = np.arange(FW, dtype=np.float32) * STRIDE
    sy = np.arange(FH, dtype=np.float32) * STRIDE
    sxx, syy = np.meshgrid(sx, sy)
    shifts = np.stack([sxx.ravel(), syy.ravel(), sxx.ravel(), syy.ravel()], axis=1)
    return (shifts[:, None, :] + base[None, :, :]).reshape(-1, 4)


_ANCHORS = jnp.asarray(_grid_anchors_np())


def _conv2d(x, w, b, pad):
    y = jax.lax.conv_general_dilated(x, w, (1, 1), [(pad, pad), (pad, pad)],
                                     dimension_numbers=('NCHW', 'OIHW', 'NCHW'))
    return y + b[None, :, None, None]


def _nms_kernel(b_ref, keep_ref):
    x1 = b_ref[0]
    y1 = b_ref[1]
    x2 = b_ref[2]
    y2 = b_ref[3]
    rows = jax.lax.broadcasted_iota(jnp.int32, (NR, NC), 0)
    cols = jax.lax.broadcasted_iota(jnp.int32, (NR, NC), 1)
    idx = rows * NC + cols
    areas = (x2 - x1 + 1) * (y2 - y1 + 1)

    def get(v, i):
        return jnp.sum(jnp.where(idx == i, v, 0.0))

    def body(i, keep):
        x1i = get(x1, i)
        y1i = get(y1, i)
        x2i = get(x2, i)
        y2i = get(y2, i)
        ai = get(areas, i)
        ki = get(keep, i)
        xx1 = jnp.maximum(x1i, x1)
        yy1 = jnp.maximum(y1i, y1)
        xx2 = jnp.minimum(x2i, x2)
        yy2 = jnp.minimum(y2i, y2)
        w = jnp.clip(xx2 - xx1 + 1, 0.0)
        h = jnp.clip(yy2 - yy1 + 1, 0.0)
        inter = w * h
        iou = inter / (ai + areas - inter)
        sup = (iou > NMS_TH) & (idx > i) & (ki > 0.5)
        return jnp.where(sup, 0.0, keep)

    keep_ref[...] = jax.lax.fori_loop(
        0, PRE_NMS, body, jnp.ones((NR, NC), jnp.float32))


@jax.jit
def kernel(features, conv_w, conv_b, cls_w, cls_b, bbox_w, bbox_b):
    t = jax.nn.relu(_conv2d(features, conv_w, conv_b, 1))
    obj = _conv2d(t, cls_w, cls_b, 0)
    breg = _conv2d(t, bbox_w, bbox_b, 0)
    obj = obj.reshape(1, A, 1, FH, FW).transpose(0, 3, 4, 1, 2).reshape(-1)
    breg = breg.reshape(1, A, 4, FH, FW).transpose(0, 3, 4, 1, 2).reshape(-1, 4)
    scores = jax.nn.sigmoid(obj)
    top_scores, top_idx = jax.lax.top_k(scores, PRE_NMS)
    a = _ANCHORS[top_idx]
    r = breg[top_idx]
    widths = a[:, 2] - a[:, 0] + 1
    heights = a[:, 3] - a[:, 1] + 1
    ctr_x = a[:, 0] + 0.5 * widths
    ctr_y = a[:, 1] + 0.5 * heights
    dx, dy = r[:, 0], r[:, 1]
    dw = jnp.minimum(r[:, 2], BBOX_XFORM_CLIP)
    dh = jnp.minimum(r[:, 3], BBOX_XFORM_CLIP)
    pcx = dx * widths + ctr_x
    pcy = dy * heights + ctr_y
    pw = jnp.exp(dw) * widths
    ph = jnp.exp(dh) * heights
    boxes = jnp.stack([pcx - 0.5 * pw, pcy - 0.5 * ph,
                       pcx + 0.5 * pw - 1, pcy + 0.5 * ph - 1], axis=1)
    boxes = jnp.stack([jnp.clip(boxes[:, 0], 0, IMG_W - 1),
                       jnp.clip(boxes[:, 1], 0, IMG_H - 1),
                       jnp.clip(boxes[:, 2], 0, IMG_W - 1),
                       jnp.clip(boxes[:, 3], 0, IMG_H - 1)], axis=1)

    bpad = jnp.pad(boxes, ((0, NPAD - PRE_NMS), (0, 0)))
    bt = bpad.T.reshape(4, NR, NC)
    keepf = pl.pallas_call(
        _nms_kernel,
        out_shape=jax.ShapeDtypeStruct((NR, NC), jnp.float32),
    )(bt)
    keep = keepf.reshape(NPAD)[:PRE_NMS] > 0.5

    sel = jnp.nonzero(keep, size=POST_NMS, fill_value=PRE_NMS - 1)[0]
    out_boxes = boxes[sel]
    out_scores = top_scores[sel]
    order = jnp.argsort(-out_scores)
    out_boxes = out_boxes[order]
    out_scores = out_scores[order]
    return jnp.concatenate([out_boxes, out_scores[:, None]], axis=1)


# speedup vs baseline: 24.0346x; 2.3781x over previous
import math

import jax
import jax.numpy as jnp
import numpy as np
from jax.experimental import pallas as pl

IN_CH = 256
STRIDE = 16
SIZES = (32, 64, 128, 256, 512)
RATIOS = (0.5, 1.0, 2.0)
A = len(SIZES) * len(RATIOS)
IMG_H = 800
IMG_W = 800
FH = 50
FW = 50
PRE_NMS = 2000
POST_NMS = 1000
NMS_TH = 0.7
BBOX_XFORM_CLIP = math.log(1000.0 / 16)

NPAD = 2048
CH = 128
NCHUNK = NPAD // CH


def _base_anchors_np():
    w = float(STRIDE)
    h = float(STRIDE)
    x_ctr = 0.5 * (w - 1)
    y_ctr = 0.5 * (h - 1)
    size = w * h
    anchors = []
    for r in RATIOS:
        ws = round(math.sqrt(size / r))
        hs = round(ws * r)
        for s in SIZES:
            sc = s / STRIDE
            W_ = ws * sc
            H_ = hs * sc
            anchors.append([x_ctr - 0.5 * (W_ - 1), y_ctr - 0.5 * (H_ - 1),
                            x_ctr + 0.5 * (W_ - 1), y_ctr + 0.5 * (H_ - 1)])
    return np.array(anchors, dtype=np.float32)


def _grid_anchors_np():
    base = _base_anchors_np()
    sx = np.arange(FW, dtype=np.float32) * STRIDE
    sy = np.arange(FH, dtype=np.float32) * STRIDE
    sxx, syy = np.meshgrid(sx, sy)
    shifts = np.stack([sxx.ravel(), syy.ravel(), sxx.ravel(), syy.ravel()], axis=1)
    return (shifts[:, None, :] + base[None, :, :]).reshape(-1, 4)


_ANCHORS = jnp.asarray(_grid_anchors_np())


def _conv2d(x, w, b, pad):
    y = jax.lax.conv_general_dilated(x, w, (1, 1), [(pad, pad), (pad, pad)],
                                     dimension_numbers=('NCHW', 'OIHW', 'NCHW'))
    return y + b[None, :, None, None]


def _nms_kernel(bcol_ref, brow_ref, keep_ref):
    x1r = brow_ref[0:1, :]
    y1r = brow_ref[1:2, :]
    x2r = brow_ref[2:3, :]
    y2r = brow_ref[3:4, :]
    arear = (x2r - x1r + 1) * (y2r - y1r + 1)
    keep = jnp.ones((1, NPAD), jnp.float32)
    for c in range(NCHUNK):
        cs = c * CH
        ce = cs + CH
        W = NPAD - cs
        x1c = bcol_ref[cs:ce, 0:1]
        y1c = bcol_ref[cs:ce, 1:2]
        x2c = bcol_ref[cs:ce, 2:3]
        y2c = bcol_ref[cs:ce, 3:4]
        areac = (x2c - x1c + 1) * (y2c - y1c + 1)
        xx1 = jnp.maximum(x1c, x1r[:, cs:])
        yy1 = jnp.maximum(y1c, y1r[:, cs:])
        xx2 = jnp.minimum(x2c, x2r[:, cs:])
        yy2 = jnp.minimum(y2c, y2r[:, cs:])
        w = jnp.clip(xx2 - xx1 + 1, 0.0)
        h = jnp.clip(yy2 - yy1 + 1, 0.0)
        inter = w * h
        iou = inter / (areac + arear[:, cs:] - inter)
        rowi = jax.lax.broadcasted_iota(jnp.int32, (CH, W), 0)
        colj = jax.lax.broadcasted_iota(jnp.int32, (CH, W), 1)
        supb = jnp.where((iou > NMS_TH) & (rowi < colj),
                         1.0, 0.0).astype(jnp.bfloat16)
        diag = supb[:, :CH]
        kinit = keep[:, cs:ce]

        def fix_cond(carry):
            return carry[1]

        def fix_body(carry, _diag=diag, _kinit=kinit):
            k, _ = carry
            m = jax.lax.dot_general(k.astype(jnp.bfloat16), _diag,
                                    (((1,), (0,)), ((), ())),
                                    preferred_element_type=jnp.float32)
            kn = jnp.where(m > 0.5, 0.0, _kinit)
            return kn, jnp.any(kn != k)

        kc, _ = jax.lax.while_loop(fix_cond, fix_body, (kinit, True))
        parts = ([keep[:, :cs]] if cs > 0 else []) + [kc]
        if W > CH:
            m = jax.lax.dot_general(kc.astype(jnp.bfloat16), supb[:, CH:],
                                    (((1,), (0,)), ((), ())),
                                    preferred_element_type=jnp.float32)
            parts.append(jnp.where(m > 0.5, 0.0, keep[:, ce:]))
        keep = jnp.concatenate(parts, axis=1) if len(parts) > 1 else parts[0]
    keep_ref[...] = keep


@jax.jit
def kernel(features, conv_w, conv_b, cls_w, cls_b, bbox_w, bbox_b):
    t = jax.nn.relu(_conv2d(features, conv_w, conv_b, 1))
    obj = _conv2d(t, cls_w, cls_b, 0)
    breg = _conv2d(t, bbox_w, bbox_b, 0)
    obj = obj.reshape(1, A, 1, FH, FW).transpose(0, 3, 4, 1, 2).reshape(-1)
    breg = breg.reshape(1, A, 4, FH, FW).transpose(0, 3, 4, 1, 2).reshape(-1, 4)
    scores = jax.nn.sigmoid(obj)
    top_scores, top_idx = jax.lax.top_k(scores, PRE_NMS)
    a = _ANCHORS[top_idx]
    r = breg[top_idx]
    widths = a[:, 2] - a[:, 0] + 1
    heights = a[:, 3] - a[:, 1] + 1
    ctr_x = a[:, 0] + 0.5 * widths
    ctr_y = a[:, 1] + 0.5 * heights
    dx, dy = r[:, 0], r[:, 1]
    dw = jnp.minimum(r[:, 2], BBOX_XFORM_CLIP)
    dh = jnp.minimum(r[:, 3], BBOX_XFORM_CLIP)
    pcx = dx * widths + ctr_x
    pcy = dy * heights + ctr_y
    pw = jnp.exp(dw) * widths
    ph = jnp.exp(dh) * heights
    boxes = jnp.stack([pcx - 0.5 * pw, pcy - 0.5 * ph,
                       pcx + 0.5 * pw - 1, pcy + 0.5 * ph - 1], axis=1)
    boxes = jnp.stack([jnp.clip(boxes[:, 0], 0, IMG_W - 1),
                       jnp.clip(boxes[:, 1], 0, IMG_H - 1),
                       jnp.clip(boxes[:, 2], 0, IMG_W - 1),
                       jnp.clip(boxes[:, 3], 0, IMG_H - 1)], axis=1)

    bpad = jnp.pad(boxes, ((0, NPAD - PRE_NMS), (0, 0)))
    keepf = pl.pallas_call(
        _nms_kernel,
        out_shape=jax.ShapeDtypeStruct((1, NPAD), jnp.float32),
    )(bpad, bpad.T)
    keep = keepf[0, :PRE_NMS] > 0.5

    sel = jnp.nonzero(keep, size=POST_NMS, fill_value=PRE_NMS - 1)[0]
    out_boxes = boxes[sel]
    out_scores = top_scores[sel]
    order = jnp.argsort(-out_scores)
    out_boxes = out_boxes[order]
    out_scores = out_scores[order]
    return jnp.concatenate([out_boxes, out_scores[:, None]], axis=1)


# speedup vs baseline: 25.8706x; 1.0764x over previous
import math

import jax
import jax.numpy as jnp
import numpy as np
from jax.experimental import pallas as pl

IN_CH = 256
STRIDE = 16
SIZES = (32, 64, 128, 256, 512)
RATIOS = (0.5, 1.0, 2.0)
A = len(SIZES) * len(RATIOS)
IMG_H = 800
IMG_W = 800
FH = 50
FW = 50
PRE_NMS = 2000
POST_NMS = 1000
NMS_TH = 0.7
BBOX_XFORM_CLIP = math.log(1000.0 / 16)

NPAD = 2048
CH = 128
NCHUNK = NPAD // CH


def _base_anchors_np():
    w = float(STRIDE)
    h = float(STRIDE)
    x_ctr = 0.5 * (w - 1)
    y_ctr = 0.5 * (h - 1)
    size = w * h
    anchors = []
    for r in RATIOS:
        ws = round(math.sqrt(size / r))
        hs = round(ws * r)
        for s in SIZES:
            sc = s / STRIDE
            W_ = ws * sc
            H_ = hs * sc
            anchors.append([x_ctr - 0.5 * (W_ - 1), y_ctr - 0.5 * (H_ - 1),
                            x_ctr + 0.5 * (W_ - 1), y_ctr + 0.5 * (H_ - 1)])
    return np.array(anchors, dtype=np.float32)


def _grid_anchors_np():
    base = _base_anchors_np()
    sx = np.arange(FW, dtype=np.float32) * STRIDE
    sy = np.arange(FH, dtype=np.float32) * STRIDE
    sxx, syy = np.meshgrid(sx, sy)
    shifts = np.stack([sxx.ravel(), syy.ravel(), sxx.ravel(), syy.ravel()], axis=1)
    return (shifts[:, None, :] + base[None, :, :]).reshape(-1, 4)


_ANCHORS = jnp.asarray(_grid_anchors_np())


def _conv2d(x, w, b, pad):
    y = jax.lax.conv_general_dilated(x, w, (1, 1), [(pad, pad), (pad, pad)],
                                     dimension_numbers=('NCHW', 'OIHW', 'NCHW'))
    return y + b[None, :, None, None]


def _nms_kernel(bcol_ref, brow_ref, keep_ref):
    x1r = brow_ref[0:1, :]
    y1r = brow_ref[1:2, :]
    x2r = brow_ref[2:3, :]
    y2r = brow_ref[3:4, :]
    arear = (x2r - x1r + 1) * (y2r - y1r + 1)
    keep = jnp.ones((1, NPAD), jnp.float32)
    for c in range(NCHUNK):
        cs = c * CH
        ce = cs + CH
        W = NPAD - cs
        x1c = bcol_ref[cs:ce, 0:1]
        y1c = bcol_ref[cs:ce, 1:2]
        x2c = bcol_ref[cs:ce, 2:3]
        y2c = bcol_ref[cs:ce, 3:4]
        areac = (x2c - x1c + 1) * (y2c - y1c + 1)
        xx1 = jnp.maximum(x1c, x1r[:, cs:])
        yy1 = jnp.maximum(y1c, y1r[:, cs:])
        xx2 = jnp.minimum(x2c, x2r[:, cs:])
        yy2 = jnp.minimum(y2c, y2r[:, cs:])
        w = jnp.clip(xx2 - xx1 + 1, 0.0)
        h = jnp.clip(yy2 - yy1 + 1, 0.0)
        inter = w * h
        iou = inter / (areac + arear[:, cs:] - inter)
        rowi = jax.lax.broadcasted_iota(jnp.int32, (CH, W), 0)
        colj = jax.lax.broadcasted_iota(jnp.int32, (CH, W), 1)
        supb = jnp.where((iou > NMS_TH) & (rowi < colj),
                         1.0, 0.0).astype(jnp.bfloat16)
        diag = supb[:, :CH]
        kinit = keep[:, cs:ce]

        def fix_cond(carry):
            return carry[1]

        def fix_body(carry, _diag=diag, _kinit=kinit):
            k, _ = carry
            m = jax.lax.dot_general(k.astype(jnp.bfloat16), _diag,
                                    (((1,), (0,)), ((), ())),
                                    preferred_element_type=jnp.float32)
            kn = jnp.where(m > 0.5, 0.0, _kinit)
            return kn, jnp.any(kn != k)

        kc, _ = jax.lax.while_loop(fix_cond, fix_body, (kinit, True))
        parts = ([keep[:, :cs]] if cs > 0 else []) + [kc]
        if W > CH:
            m = jax.lax.dot_general(kc.astype(jnp.bfloat16), supb[:, CH:],
                                    (((1,), (0,)), ((), ())),
                                    preferred_element_type=jnp.float32)
            parts.append(jnp.where(m > 0.5, 0.0, keep[:, ce:]))
        keep = jnp.concatenate(parts, axis=1) if len(parts) > 1 else parts[0]
    keep_ref[...] = keep


@jax.jit
def kernel(features, conv_w, conv_b, cls_w, cls_b, bbox_w, bbox_b):
    t = jax.nn.relu(_conv2d(features, conv_w, conv_b, 1))
    obj = _conv2d(t, cls_w, cls_b, 0)
    breg = _conv2d(t, bbox_w, bbox_b, 0)
    obj = obj.reshape(1, A, 1, FH, FW).transpose(0, 3, 4, 1, 2).reshape(-1)
    breg = breg.reshape(1, A, 4, FH, FW).transpose(0, 3, 4, 1, 2).reshape(-1, 4)
    scores = jax.nn.sigmoid(obj)
    top_scores, top_idx = jax.lax.top_k(scores, PRE_NMS)
    a = _ANCHORS[top_idx]
    r = breg[top_idx]
    widths = a[:, 2] - a[:, 0] + 1
    heights = a[:, 3] - a[:, 1] + 1
    ctr_x = a[:, 0] + 0.5 * widths
    ctr_y = a[:, 1] + 0.5 * heights
    dx, dy = r[:, 0], r[:, 1]
    dw = jnp.minimum(r[:, 2], BBOX_XFORM_CLIP)
    dh = jnp.minimum(r[:, 3], BBOX_XFORM_CLIP)
    pcx = dx * widths + ctr_x
    pcy = dy * heights + ctr_y
    pw = jnp.exp(dw) * widths
    ph = jnp.exp(dh) * heights
    boxes = jnp.stack([pcx - 0.5 * pw, pcy - 0.5 * ph,
                       pcx + 0.5 * pw - 1, pcy + 0.5 * ph - 1], axis=1)
    boxes = jnp.stack([jnp.clip(boxes[:, 0], 0, IMG_W - 1),
                       jnp.clip(boxes[:, 1], 0, IMG_H - 1),
                       jnp.clip(boxes[:, 2], 0, IMG_W - 1),
                       jnp.clip(boxes[:, 3], 0, IMG_H - 1)], axis=1)

    bpad = jnp.pad(boxes, ((0, NPAD - PRE_NMS), (0, 0)))
    keepf = pl.pallas_call(
        _nms_kernel,
        out_shape=jax.ShapeDtypeStruct((1, NPAD), jnp.float32),
    )(bpad, bpad.T)
    keep = keepf[0, :PRE_NMS] > 0.5

    sel = jnp.nonzero(keep, size=POST_NMS, fill_value=PRE_NMS - 1)[0]
    out_boxes = boxes[sel]
    out_scores = top_scores[sel]
    return jnp.concatenate([out_boxes, out_scores[:, None]], axis=1)
